# baseline (speedup 1.0000x reference)
import functools

import jax
import jax.numpy as jnp
from jax import lax
from jax.experimental import pallas as pl
from jax.experimental.pallas import tpu as pltpu
from jax.experimental.pallas import tpu_sc as plsc

_B = 1024
_E = 32
_V = 100000
_BN = 2048


@functools.lru_cache(maxsize=None)
def _make_sc_gather(num_cores: int, num_subcores: int):
    nw = num_cores * num_subcores
    b_per_w = _B // nw
    mesh = plsc.VectorSubcoreMesh(core_axis_name="c", subcore_axis_name="s")

    @functools.partial(
        pl.kernel,
        mesh=mesh,
        out_type=jax.ShapeDtypeStruct((_B, _E), jnp.float32),
        scratch_types=[
            pltpu.VMEM((b_per_w,), jnp.int32),
            pltpu.VMEM((b_per_w, _E), jnp.float32),
            pltpu.SemaphoreType.DMA,
        ],
        compiler_params=pltpu.CompilerParams(use_tc_tiling_on_sc=False),
    )
    def gather(table_hbm, idx_hbm, out_hbm, idx_v, rows_v, sem):
        wid = lax.axis_index("s") * num_cores + lax.axis_index("c")
        base = wid * b_per_w
        pltpu.sync_copy(idx_hbm.at[pl.ds(base, b_per_w)], idx_v)
        pltpu.async_copy(table_hbm.at[idx_v], rows_v, sem).wait()
        pltpu.sync_copy(rows_v, out_hbm.at[pl.ds(base, b_per_w)])

    return gather


_BM = 16
_NBUF = 4


def _mm_body(emb_ref, w_ref, b_ref, out_ref, sbuf, sems):
    nchunks = _B // _BM

    def out_copy(c, d):
        return pltpu.make_async_copy(
            sbuf.at[d], out_ref.at[pl.ds(c * _BM, _BM)], sems.at[d]
        )

    for c in range(nchunks):
        d = c % _NBUF
        if c >= _NBUF:
            out_copy(c - _NBUF, d).wait()
        sbuf[d, :, :] = (
            jnp.dot(
                emb_ref[pl.ds(c * _BM, _BM), :],
                w_ref[...],
                preferred_element_type=jnp.float32,
            )
            + b_ref[...]
        )
        out_copy(c, d).start()
    for c in range(nchunks - _NBUF, nchunks):
        out_copy(c, c % _NBUF).wait()


def _tc_project(emb, dense_kernel, bias2d):
    return pl.pallas_call(
        _mm_body,
        out_shape=jax.ShapeDtypeStruct((_B, _V), jnp.float32),
        in_specs=[
            pl.BlockSpec(memory_space=pltpu.VMEM),
            pl.BlockSpec(memory_space=pltpu.VMEM),
            pl.BlockSpec(memory_space=pltpu.VMEM),
        ],
        out_specs=pl.BlockSpec(memory_space=pl.ANY),
        scratch_shapes=[
            pltpu.VMEM((_NBUF, _BM, _V), jnp.float32),
            pltpu.SemaphoreType.DMA((_NBUF,)),
        ],
    )(emb, dense_kernel, bias2d)


def kernel(function_type_ids, embedding_table, dense_kernel, dense_bias):
    info = plsc.get_sparse_core_info()
    ids = function_type_ids.astype(jnp.int32)
    emb = _make_sc_gather(info.num_cores, info.num_subcores)(
        embedding_table, ids
    )
    return _tc_project(emb, dense_kernel, dense_bias.reshape(1, _V))


# speedup vs baseline: 2.1357x; 2.1357x over previous
import functools

import jax
import jax.numpy as jnp
from jax import lax
from jax.experimental import pallas as pl
from jax.experimental.pallas import tpu as pltpu
from jax.experimental.pallas import tpu_sc as plsc

_B = 1024
_E = 32
_V = 100000
_BN = 2048


@functools.lru_cache(maxsize=None)
def _make_sc_gather(num_cores: int, num_subcores: int):
    nw = num_cores * num_subcores
    b_per_w = _B // nw
    mesh = plsc.VectorSubcoreMesh(core_axis_name="c", subcore_axis_name="s")

    @functools.partial(
        pl.kernel,
        mesh=mesh,
        out_type=jax.ShapeDtypeStruct((_B, _E), jnp.float32),
        scratch_types=[
            pltpu.VMEM((b_per_w,), jnp.int32),
            pltpu.VMEM((b_per_w, _E), jnp.float32),
            pltpu.SemaphoreType.DMA,
        ],
        compiler_params=pltpu.CompilerParams(use_tc_tiling_on_sc=False),
    )
    def gather(table_hbm, idx_hbm, out_hbm, idx_v, rows_v, sem):
        wid = lax.axis_index("s") * num_cores + lax.axis_index("c")
        base = wid * b_per_w
        pltpu.sync_copy(idx_hbm.at[pl.ds(base, b_per_w)], idx_v)
        pltpu.async_copy(table_hbm.at[idx_v], rows_v, sem).wait()
        pltpu.sync_copy(rows_v, out_hbm.at[pl.ds(base, b_per_w)])

    return gather


def _mm_body(w_ref, emb_ref, b_ref, out_ref):
    out_ref[...] = (
        jax.lax.dot_general(
            w_ref[...],
            emb_ref[...],
            (((0,), (1,)), ((), ())),
            preferred_element_type=jnp.float32,
        )
        + b_ref[...]
    )


def _tc_project(emb, dense_kernel, bias_col):
    grid = pl.cdiv(_V, _BN)
    out_t = pl.pallas_call(
        _mm_body,
        out_shape=jax.ShapeDtypeStruct((_V, _B), jnp.float32),
        grid=(grid,),
        in_specs=[
            pl.BlockSpec((_E, _BN), lambda i: (0, i)),
            pl.BlockSpec((_B, _E), lambda i: (0, 0)),
            pl.BlockSpec((_BN, 1), lambda i: (i, 0)),
        ],
        out_specs=pl.BlockSpec((_BN, _B), lambda i: (i, 0)),
        compiler_params=pltpu.CompilerParams(
            dimension_semantics=("parallel",),
        ),
    )(dense_kernel, emb, bias_col)
    return out_t.T


def kernel(function_type_ids, embedding_table, dense_kernel, dense_bias):
    info = plsc.get_sparse_core_info()
    ids = function_type_ids.astype(jnp.int32)
    emb = _make_sc_gather(info.num_cores, info.num_subcores)(
        embedding_table, ids
    )
    return _tc_project(emb, dense_kernel, dense_bias.reshape(_V, 1))


# speedup vs baseline: 2.7341x; 1.2802x over previous
import functools

import jax
import jax.numpy as jnp
from jax import lax
from jax.experimental import pallas as pl
from jax.experimental.pallas import tpu as pltpu
from jax.experimental.pallas import tpu_sc as plsc

_B = 1024
_E = 32
_V = 100000
_BN = 2048
_R = 128 // _E


@functools.lru_cache(maxsize=None)
def _make_sc_gather(num_cores: int, num_subcores: int):
    nw = num_cores * num_subcores
    b_per_w = _B // nw
    mesh = plsc.VectorSubcoreMesh(core_axis_name="c", subcore_axis_name="s")

    @functools.partial(
        pl.kernel,
        mesh=mesh,
        out_type=jax.ShapeDtypeStruct((_B, 128), jnp.float32),
        scratch_types=[
            pltpu.VMEM((b_per_w,), jnp.int32),
            pltpu.VMEM((b_per_w, 128), jnp.float32),
            pltpu.SemaphoreType.DMA,
        ],
    )
    def gather(table_hbm, idx_hbm, out_hbm, idx_v, rows_v, sem):
        wid = lax.axis_index("s") * num_cores + lax.axis_index("c")
        base = wid * b_per_w
        pltpu.sync_copy(idx_hbm.at[pl.ds(base, b_per_w)], idx_v)
        pltpu.async_copy(table_hbm.at[idx_v], rows_v, sem).wait()
        pltpu.sync_copy(rows_v, out_hbm.at[pl.ds(base, b_per_w)])

    return gather


def _mm_body(w_ref, be_ref, rem_ref, b_ref, out_ref):
    rem = rem_ref[...]
    big = be_ref[...]
    emb = jnp.where(rem == 0, big[:, 0:_E], 0.0)
    for q in range(1, _R):
        emb = emb + jnp.where(rem == q, big[:, q * _E:(q + 1) * _E], 0.0)
    w_ext = jnp.concatenate([w_ref[...], b_ref[...]], axis=0)
    emb_ext = jnp.concatenate(
        [emb, jnp.ones((_B, 1), jnp.float32)], axis=1
    )
    out_ref[...] = lax.dot_general(
        w_ext,
        emb_ext,
        (((0,), (1,)), ((), ())),
        preferred_element_type=jnp.float32,
    )


def _tc_project(big_emb, rem_col, dense_kernel, bias_row):
    grid = pl.cdiv(_V, _BN)
    out_t = pl.pallas_call(
        _mm_body,
        out_shape=jax.ShapeDtypeStruct((_V, _B), jnp.float32),
        grid=(grid,),
        in_specs=[
            pl.BlockSpec((_E, _BN), lambda i: (0, i)),
            pl.BlockSpec((_B, 128), lambda i: (0, 0)),
            pl.BlockSpec((_B, 1), lambda i: (0, 0)),
            pl.BlockSpec((1, _BN), lambda i: (0, i)),
        ],
        out_specs=pl.BlockSpec((_BN, _B), lambda i: (i, 0)),
        compiler_params=pltpu.CompilerParams(
            dimension_semantics=("parallel",),
        ),
    )(dense_kernel, big_emb, rem_col, bias_row)
    return out_t.T


def kernel(function_type_ids, embedding_table, dense_kernel, dense_bias):
    info = plsc.get_sparse_core_info()
    ids = function_type_ids.astype(jnp.int32)
    big_idx = ids // _R
    rem_col = (ids % _R).reshape(_B, 1)
    table128 = embedding_table.reshape(_V // _R, 128)
    big_emb = _make_sc_gather(info.num_cores, info.num_subcores)(
        table128, big_idx
    )
    return _tc_project(
        big_emb, rem_col, dense_kernel, dense_bias.reshape(1, _V)
    )


# speedup vs baseline: 2.8382x; 1.0381x over previous
import functools

import jax
import jax.numpy as jnp
from jax import lax
from jax.experimental import pallas as pl
from jax.experimental.pallas import tpu as pltpu
from jax.experimental.pallas import tpu_sc as plsc

_B = 1024
_E = 32
_V = 100000
_BN = 4096
_R = 128 // _E


@functools.lru_cache(maxsize=None)
def _make_sc_gather(num_cores: int, num_subcores: int):
    nw = num_cores * num_subcores
    b_per_w = _B // nw
    mesh = plsc.VectorSubcoreMesh(core_axis_name="c", subcore_axis_name="s")

    @functools.partial(
        pl.kernel,
        mesh=mesh,
        out_type=jax.ShapeDtypeStruct((_B, 128), jnp.float32),
        scratch_types=[
            pltpu.VMEM((b_per_w,), jnp.int32),
            pltpu.VMEM((b_per_w, 128), jnp.float32),
            pltpu.SemaphoreType.DMA,
        ],
    )
    def gather(table_hbm, idx_hbm, out_hbm, idx_v, rows_v, sem):
        wid = lax.axis_index("s") * num_cores + lax.axis_index("c")
        base = wid * b_per_w
        pltpu.sync_copy(idx_hbm.at[pl.ds(base, b_per_w)], idx_v)
        pltpu.async_copy(table_hbm.at[idx_v], rows_v, sem).wait()
        pltpu.sync_copy(rows_v, out_hbm.at[pl.ds(base, b_per_w)])

    return gather


def _mm_body(w_ref, be_ref, b_ref, out_ref):
    emb = be_ref[...][:, 0:_E]
    w_ext = jnp.concatenate([w_ref[...], b_ref[...]], axis=0)
    emb_ext = jnp.concatenate(
        [emb, jnp.ones((_B, 1), jnp.float32)], axis=1
    )
    out_ref[...] = lax.dot_general(
        w_ext,
        emb_ext,
        (((0,), (1,)), ((), ())),
        preferred_element_type=jnp.float32,
    )


def _tc_project(big_emb, dense_kernel, bias_row):
    grid = pl.cdiv(_V, _BN)
    out_t = pl.pallas_call(
        _mm_body,
        out_shape=jax.ShapeDtypeStruct((_V, _B), jnp.float32),
        grid=(grid,),
        in_specs=[
            pl.BlockSpec((_E, _BN), lambda i: (0, i)),
            pl.BlockSpec((_B, 128), lambda i: (0, 0)),
            pl.BlockSpec((1, _BN), lambda i: (0, i)),
        ],
        out_specs=pl.BlockSpec((_BN, _B), lambda i: (i, 0)),
        compiler_params=pltpu.CompilerParams(
            dimension_semantics=("parallel",),
        ),
    )(dense_kernel, big_emb, bias_row)
    return out_t.T


def kernel(function_type_ids, embedding_table, dense_kernel, dense_bias):
    info = plsc.get_sparse_core_info()
    ids = function_type_ids.astype(jnp.int32)
    table_pad = jnp.pad(embedding_table, ((0, 0), (0, 128 - _E)))
    emb_pad = _make_sc_gather(info.num_cores, info.num_subcores)(
        table_pad, ids
    )
    return _tc_project(emb_pad, dense_kernel, dense_bias.reshape(1, _V))


# speedup vs baseline: 3.2268x; 1.1370x over previous
import functools

import jax
import jax.numpy as jnp
from jax import lax
from jax.experimental import pallas as pl
from jax.experimental.pallas import tpu as pltpu
from jax.experimental.pallas import tpu_sc as plsc

_B = 1024
_E = 32
_V = 100000
_BN = 4096
_R = 128 // _E


@functools.lru_cache(maxsize=None)
def _make_sc_gather(num_cores: int, num_subcores: int):
    nw = num_cores * num_subcores
    b_per_w = _B // nw
    mesh = plsc.VectorSubcoreMesh(core_axis_name="c", subcore_axis_name="s")

    @functools.partial(
        pl.kernel,
        mesh=mesh,
        out_type=jax.ShapeDtypeStruct((_B, 128), jnp.float32),
        scratch_types=[
            pltpu.VMEM((b_per_w,), jnp.int32),
            pltpu.VMEM((b_per_w, 128), jnp.float32),
            pltpu.SemaphoreType.DMA,
        ],
    )
    def gather(table_hbm, idx_hbm, out_hbm, idx_v, rows_v, sem):
        wid = lax.axis_index("s") * num_cores + lax.axis_index("c")
        base = wid * b_per_w
        pltpu.sync_copy(idx_hbm.at[pl.ds(base, b_per_w)], idx_v)
        pltpu.async_copy(table_hbm.at[idx_v], rows_v, sem).wait()
        pltpu.sync_copy(rows_v, out_hbm.at[pl.ds(base, b_per_w)])

    return gather


_BC = 8192


def _tp_body(tT_ref, out_ref):
    out_ref[:, 0:_E] = tT_ref[...].T


def _transpose_pad(table_t):
    grid = pl.cdiv(_V, _BC)
    return pl.pallas_call(
        _tp_body,
        out_shape=jax.ShapeDtypeStruct((_V, 128), jnp.float32),
        grid=(grid,),
        in_specs=[pl.BlockSpec((_E, _BC), lambda i: (0, i))],
        out_specs=pl.BlockSpec((_BC, 128), lambda i: (i, 0)),
        compiler_params=pltpu.CompilerParams(
            dimension_semantics=("parallel",),
        ),
    )(table_t)


def _mm_body(w_ref, be_ref, b_ref, out_ref):
    emb = be_ref[...][:, 0:_E]
    w_ext = jnp.concatenate([w_ref[...], b_ref[...]], axis=0)
    emb_ext = jnp.concatenate(
        [emb, jnp.ones((_B, 1), jnp.float32)], axis=1
    )
    out_ref[...] = lax.dot_general(
        w_ext,
        emb_ext,
        (((0,), (1,)), ((), ())),
        preferred_element_type=jnp.float32,
    )


def _tc_project(big_emb, dense_kernel, bias_row):
    grid = pl.cdiv(_V, _BN)
    out_t = pl.pallas_call(
        _mm_body,
        out_shape=jax.ShapeDtypeStruct((_V, _B), jnp.float32),
        grid=(grid,),
        in_specs=[
            pl.BlockSpec((_E, _BN), lambda i: (0, i)),
            pl.BlockSpec((_B, 128), lambda i: (0, 0)),
            pl.BlockSpec((1, _BN), lambda i: (0, i)),
        ],
        out_specs=pl.BlockSpec((_BN, _B), lambda i: (i, 0)),
        compiler_params=pltpu.CompilerParams(
            dimension_semantics=("parallel",),
        ),
    )(dense_kernel, big_emb, bias_row)
    return out_t.T


def kernel(function_type_ids, embedding_table, dense_kernel, dense_bias):
    info = plsc.get_sparse_core_info()
    ids = function_type_ids.astype(jnp.int32)
    table_pad = _transpose_pad(embedding_table.T)
    emb_pad = _make_sc_gather(info.num_cores, info.num_subcores)(
        table_pad, ids
    )
    return _tc_project(emb_pad, dense_kernel, dense_bias.reshape(1, _V))


# speedup vs baseline: 3.2750x; 1.0149x over previous
import functools

import jax
import jax.numpy as jnp
from jax import lax
from jax.experimental import pallas as pl
from jax.experimental.pallas import tpu as pltpu
from jax.experimental.pallas import tpu_sc as plsc

_B = 1024
_E = 32
_V = 100000
_BN = 4096
_R = 128 // _E


@functools.lru_cache(maxsize=None)
def _make_sc_gather(num_cores: int, num_subcores: int):
    nw = num_cores * num_subcores
    b_per_w = _B // nw
    mesh = plsc.VectorSubcoreMesh(core_axis_name="c", subcore_axis_name="s")

    @functools.partial(
        pl.kernel,
        mesh=mesh,
        out_type=jax.ShapeDtypeStruct((_B, 128), jnp.float32),
        scratch_types=[
            pltpu.VMEM((b_per_w,), jnp.int32),
            pltpu.VMEM((b_per_w, 128), jnp.float32),
            pltpu.SemaphoreType.DMA,
        ],
    )
    def gather(table_hbm, idx_hbm, out_hbm, idx_v, rows_v, sem):
        wid = lax.axis_index("s") * num_cores + lax.axis_index("c")
        base = wid * b_per_w
        pltpu.sync_copy(idx_hbm.at[pl.ds(base, b_per_w)], idx_v)
        pltpu.async_copy(table_hbm.at[idx_v], rows_v, sem).wait()
        pltpu.sync_copy(rows_v, out_hbm.at[pl.ds(base, b_per_w)])

    return gather


_BC = 16384


def _tp_body(tT_ref, out_ref):
    out_ref[:, 0:_E] = tT_ref[...].T


def _transpose_pad(table_t):
    grid = pl.cdiv(_V, _BC)
    return pl.pallas_call(
        _tp_body,
        out_shape=jax.ShapeDtypeStruct((_V, 128), jnp.float32),
        grid=(grid,),
        in_specs=[pl.BlockSpec((_E, _BC), lambda i: (0, i))],
        out_specs=pl.BlockSpec((_BC, 128), lambda i: (i, 0)),
        compiler_params=pltpu.CompilerParams(
            dimension_semantics=("parallel",),
        ),
    )(table_t)


def _mm_body(w_ref, be_ref, b_ref, out_ref):
    emb = be_ref[...][:, 0:_E]
    w_ext = jnp.concatenate([w_ref[...], b_ref[...]], axis=0)
    emb_ext = jnp.concatenate(
        [emb, jnp.ones((_B, 1), jnp.float32)], axis=1
    )
    out_ref[...] = lax.dot_general(
        w_ext,
        emb_ext,
        (((0,), (1,)), ((), ())),
        preferred_element_type=jnp.float32,
    )


def _tc_project(big_emb, dense_kernel, bias_row):
    grid = pl.cdiv(_V, _BN)
    out_t = pl.pallas_call(
        _mm_body,
        out_shape=jax.ShapeDtypeStruct((_V, _B), jnp.float32),
        grid=(grid,),
        in_specs=[
            pl.BlockSpec((_E, _BN), lambda i: (0, i)),
            pl.BlockSpec((_B, 128), lambda i: (0, 0)),
            pl.BlockSpec((1, _BN), lambda i: (0, i)),
        ],
        out_specs=pl.BlockSpec((_BN, _B), lambda i: (i, 0)),
        compiler_params=pltpu.CompilerParams(
            dimension_semantics=("parallel",),
        ),
    )(dense_kernel, big_emb, bias_row)
    return out_t.T


def kernel(function_type_ids, embedding_table, dense_kernel, dense_bias):
    info = plsc.get_sparse_core_info()
    ids = function_type_ids.astype(jnp.int32)
    table_pad = _transpose_pad(embedding_table.T)
    emb_pad = _make_sc_gather(info.num_cores, info.num_subcores)(
        table_pad, ids
    )
    return _tc_project(emb_pad, dense_kernel, dense_bias.reshape(1, _V))
